# Initial kernel scaffold; baseline (speedup 1.0000x reference)
import jax
import jax.numpy as jnp
from jax.experimental import pallas as pl


def kernel(input_atom, input_bond, atom_graph, bond_graph, num_nbs, node_mask, extra, W_af, W_na, W_nb, W_sa, W_u2, b_u2, W_u1, b_u1):
    raise NotImplementedError("write your pallas kernel here")


# speedup vs baseline: 1.0071x; 1.0071x over previous
import functools

import jax
import jax.numpy as jnp
from jax import lax
from jax.experimental import pallas as pl
from jax.experimental.pallas import tpu as pltpu
from jax.experimental.pallas import tpu_sc as plsc

H = 128
MAX_NB = 10
DEPTH = 3
NC, NS = 2, 16
NW = NC * NS
BLK = 512
NEG = -1.0e9


def _tc_stage(xs, ws, bias, do_relu, extra_ws, out_sents, nblk, pad_out):
    nx, ne = len(xs), len(extra_ws)
    grid = (nblk + 1,) if pad_out else (nblk,)
    xmap = lambda i: (jnp.minimum(i, nblk - 1), 0)
    wmap = lambda i: (0, 0)
    in_specs = (
        [pl.BlockSpec((BLK, x.shape[1]), xmap) for x in xs]
        + [pl.BlockSpec(w.shape, wmap) for w in ws]
        + ([pl.BlockSpec(bias.shape, wmap)] if bias is not None else [])
        + [pl.BlockSpec(w.shape, wmap) for w in extra_ws]
    )
    r_out = (nblk + 1) * BLK if pad_out else nblk * BLK
    out_specs = [pl.BlockSpec((BLK, H), lambda i: (i, 0)) for _ in range(1 + ne)]
    out_shape = [jax.ShapeDtypeStruct((r_out, H), jnp.float32) for _ in range(1 + ne)]

    def body(*refs):
        xr = refs[:nx]
        wr = refs[nx:2 * nx]
        k = 2 * nx
        br = None
        if bias is not None:
            br = refs[k]
            k += 1
        ewr = refs[k:k + ne]
        outr = refs[k + ne:]
        y = jnp.dot(xr[0][...], wr[0][...], preferred_element_type=jnp.float32)
        for t in range(1, nx):
            y = y + jnp.dot(xr[t][...], wr[t][...], preferred_element_type=jnp.float32)
        if br is not None:
            y = y + br[...]
        if do_relu:
            y = jnp.maximum(y, 0.0)
        outs = [y] + [jnp.dot(y, er[...], preferred_element_type=jnp.float32) for er in ewr]
        if pad_out:
            i = pl.program_id(0)

            @pl.when(i < nblk)
            def _():
                for o_ref, o in zip(outr, outs):
                    o_ref[...] = o

            @pl.when(i == nblk)
            def _():
                for o_ref, sv in zip(outr, out_sents):
                    o_ref[...] = jnp.full((BLK, H), sv, jnp.float32)
        else:
            for o_ref, o in zip(outr, outs):
                o_ref[...] = o

    args = list(xs) + list(ws) + ([bias] if bias is not None else []) + list(extra_ws)
    res = pl.pallas_call(body, grid=grid, in_specs=in_specs,
                         out_specs=out_specs, out_shape=out_shape)(*args)
    return res


def _tc_bond(x, w_bu, w_bn, b2, nblk):
    xmap = lambda i: (jnp.minimum(i, nblk - 1), 0)
    wmap = lambda i: (0, 0)

    def body(x_ref, wbu_ref, wbn_ref, b_ref, bu_ref, bn_ref):
        xv = x_ref[...]
        bu = jnp.dot(xv, wbu_ref[...], preferred_element_type=jnp.float32) + b_ref[...]
        bn = jnp.dot(xv, wbn_ref[...], preferred_element_type=jnp.float32)
        i = pl.program_id(0)

        @pl.when(i < nblk)
        def _():
            bu_ref[...] = bu
            bn_ref[...] = bn

        @pl.when(i == nblk)
        def _():
            bu_ref[...] = jnp.full((BLK, H), NEG, jnp.float32)
            bn_ref[...] = jnp.zeros((BLK, H), jnp.float32)

    r_out = (nblk + 1) * BLK
    return pl.pallas_call(
        body, grid=(nblk + 1,),
        in_specs=[pl.BlockSpec((BLK, x.shape[1]), xmap),
                  pl.BlockSpec(w_bu.shape, wmap),
                  pl.BlockSpec(w_bn.shape, wmap),
                  pl.BlockSpec(b2.shape, wmap)],
        out_specs=[pl.BlockSpec((BLK, H), lambda i: (i, 0))] * 2,
        out_shape=[jax.ShapeDtypeStruct((r_out, H), jnp.float32)] * 2,
    )(x, w_bu, w_bn, b2)


def _tc_final(af, nl, prod, fs, nm2, w1a, w1b, b1, nblk):
    xmap = lambda i: (i, 0)
    wmap = lambda i: (0, 0)

    def body(af_ref, nl_ref, pr_ref, fs_ref, nm_ref, w1a_ref, w1b_ref, b_ref,
             out1_ref, af3_ref):
        out1_ref[...] = pr_ref[...] * fs_ref[...] * nm_ref[...]
        y = jnp.dot(af_ref[...], w1a_ref[...], preferred_element_type=jnp.float32)
        y = y + jnp.dot(nl_ref[...], w1b_ref[...], preferred_element_type=jnp.float32)
        af3_ref[...] = jnp.maximum(y + b_ref[...], 0.0)

    r_out = nblk * BLK
    return pl.pallas_call(
        body, grid=(nblk,),
        in_specs=[pl.BlockSpec((BLK, H), xmap)] * 4
        + [pl.BlockSpec((BLK, 1), xmap),
           pl.BlockSpec(w1a.shape, wmap), pl.BlockSpec(w1b.shape, wmap),
           pl.BlockSpec(b1.shape, wmap)],
        out_specs=[pl.BlockSpec((BLK, H), lambda i: (i, 0))] * 2,
        out_shape=[jax.ShapeDtypeStruct((r_out, H), jnp.float32)] * 2,
    )(af, nl, prod, fs, nm2, w1a, w1b, b1)


def _sc_pass(tab_a, tab_b, aidx, bidx, n_atoms, out_rows, is_relu):
    apw = n_atoms // NW
    ca = 8
    rpc = ca * MAX_NB
    nchunk = apw // ca
    mesh = plsc.VectorSubcoreMesh(core_axis_name="c", subcore_axis_name="s")

    @functools.partial(
        pl.kernel, mesh=mesh,
        out_type=jax.ShapeDtypeStruct((out_rows, H), jnp.float32),
        scratch_types=[
            pltpu.VMEM((rpc,), jnp.int32),
            pltpu.VMEM((rpc,), jnp.int32),
            pltpu.VMEM((rpc, H), jnp.float32),
            pltpu.VMEM((rpc, H), jnp.float32),
            pltpu.VMEM((ca, H), jnp.float32),
            pltpu.SemaphoreType.DMA,
            pltpu.SemaphoreType.DMA,
        ],
    )
    def k(ta, tb, ia, ib, out, ia_v, ib_v, ra_v, rb_v, ob_v, s1, s2):
        wid = lax.axis_index("s") * NC + lax.axis_index("c")
        a0 = wid * apw

        def chunk(c, carry):
            ab = a0 + c * ca
            r0 = ab * MAX_NB
            pltpu.sync_copy(ia.at[pl.ds(r0, rpc)], ia_v)
            pltpu.sync_copy(ib.at[pl.ds(r0, rpc)], ib_v)
            cpa = pltpu.async_copy(ta.at[ia_v], ra_v, s1)
            cpb = pltpu.async_copy(tb.at[ib_v], rb_v, s2)
            cpa.wait()
            cpb.wait()

            def atom(i, carry2):
                base = i * MAX_NB
                for j in range(H // 16):
                    sl = pl.ds(j * 16, 16)
                    acc = None
                    for kk in range(MAX_NB):
                        a_ = ra_v[base + kk, sl]
                        b_ = rb_v[base + kk, sl]
                        t = jnp.maximum(a_ + b_, 0.0) if is_relu else a_ * b_
                        acc = t if acc is None else acc + t
                    ob_v[i, sl] = acc
                return carry2

            lax.fori_loop(0, ca, atom, 0)
            pltpu.sync_copy(ob_v, out.at[pl.ds(ab, ca)])
            return carry

        lax.fori_loop(0, nchunk, chunk, 0)

    return k(tab_a, tab_b, aidx, bidx)


def kernel(input_atom, input_bond, atom_graph, bond_graph, num_nbs, node_mask,
           extra, W_af, W_na, W_nb, W_sa, W_u2, b_u2, W_u1, b_u1):
    B, N, F_A = input_atom.shape
    NBb = input_bond.shape[1]
    F_B = input_bond.shape[2]
    A = B * N
    Bn = B * NBb
    nblk_a = A // BLK
    nblk_b = Bn // BLK

    ag = atom_graph.astype(jnp.int32)
    bg = bond_graph.astype(jnp.int32)
    a_flat = ag[..., 0] * N + ag[..., 1]
    b_flat = bg[..., 0] * NBb + bg[..., 1]
    valid = jnp.arange(MAX_NB, dtype=jnp.int32)[None, None, :] < num_nbs.astype(jnp.int32)[:, :, None]
    aidx = jnp.where(valid, a_flat, A).reshape(-1)
    bidx = jnp.where(valid, b_flat, Bn).reshape(-1)

    fa_pad = (-F_A) % 8
    fb_pad = (-F_B) % 8
    x_atom = jnp.pad(input_atom.reshape(A, F_A), ((0, 0), (0, fa_pad)))
    x_bond = jnp.pad(input_bond.reshape(Bn, F_B), ((0, 0), (0, fb_pad)))
    W_af_p = jnp.pad(W_af, ((0, fa_pad), (0, 0)))
    W_u2a = W_u2[:H]
    W_u2b = jnp.pad(W_u2[H:], ((0, fb_pad), (0, 0)))
    W_nb_p = jnp.pad(W_nb, ((0, fb_pad), (0, 0)))
    W_u1a, W_u1b = W_u1[:H], W_u1[H:]
    b1 = b_u1.reshape(1, H)
    b2 = b_u2.reshape(1, H)
    nm2 = node_mask.reshape(A, 1)

    BU, BN_t = _tc_bond(x_bond, W_u2b, W_nb_p, b2, nblk_b)

    AF, AU = _tc_stage([x_atom], [W_af_p], None, False, [W_u2a],
                       [0.0, 0.0], nblk_a, True)
    for d in range(DEPTH - 1):
        NL = _sc_pass(AU, BU, aidx, bidx, A, A + BLK, True)
        last = d == DEPTH - 2
        extra_ws = [W_u2a, W_na, W_sa] if last else [W_u2a]
        sents = [0.0] * (len(extra_ws) + 1)
        res = _tc_stage([AF, NL], [W_u1a, W_u1b], b1, True, extra_ws,
                        sents, nblk_a, True)
        if last:
            AF, AU, AN, FS = res
        else:
            AF, AU = res

    NL = _sc_pass(AU, BU, aidx, bidx, A, A + BLK, True)
    prod = _sc_pass(AN, BN_t, aidx, bidx, A, A + BLK, False)
    out1, AF3 = _tc_final(AF, NL, prod, FS, nm2, W_u1a, W_u1b, b1, nblk_a)

    return out1.reshape(B, N, H), AF3.reshape(B, N, H)
